# baseline (speedup 1.0000x reference)
import functools

import jax
import jax.numpy as jnp
from jax import lax
from jax.experimental import pallas as pl
from jax.experimental.pallas import tpu as pltpu
from jax.experimental.pallas import tpu_sc as plsc

NUM_SEGMENTS = 128
D_FEAT = 128
N_ROWS = 100000
WIN = 80
NUM_WINDOWS = N_ROWS // WIN
NC = 2
NS = 16
NW = NC * NS
SLOTS = (NUM_WINDOWS + NW - 1) // NW
NVEC = D_FEAT // 16


def _sc_segment_sums(feat, ids2d):
    mesh = plsc.VectorSubcoreMesh(core_axis_name="c", subcore_axis_name="s")

    @functools.partial(
        pl.kernel,
        out_type=(
            jax.ShapeDtypeStruct((NW, NUM_SEGMENTS, D_FEAT), jnp.float32),
            jax.ShapeDtypeStruct((NW, NUM_SEGMENTS), jnp.float32),
        ),
        mesh=mesh,
        compiler_params=pltpu.CompilerParams(
            use_tc_tiling_on_sc=False, needs_layout_passes=False),
        scratch_types=[
            pltpu.VMEM((WIN, D_FEAT), jnp.float32),
            pltpu.VMEM((WIN, D_FEAT), jnp.float32),
            pltpu.VMEM((SLOTS, WIN), jnp.int32),
            pltpu.VMEM((NUM_SEGMENTS,), jnp.float32),
            pltpu.VMEM((NUM_SEGMENTS, D_FEAT), jnp.float32),
            pltpu.SemaphoreType.DMA,
            pltpu.SemaphoreType.DMA,
        ],
    )
    def seg_sum(feat_hbm, ids_hbm, out_sum, out_cnt,
                fbuf_a, fbuf_b, idx_all, cnt_buf, acc, sem_a, sem_b):
        c = lax.axis_index("c")
        s = lax.axis_index("s")
        w = s * NC + c

        n_w = jnp.where(w < 2, SLOTS, SLOTS - 1)
        b_w = (SLOTS - 1) * w + jnp.minimum(w, 2)
        pb = jnp.minimum(b_w, NUM_WINDOWS - SLOTS)
        shift = b_w - pb

        def win_base(l):
            return jnp.minimum(b_w + l, NUM_WINDOWS - 1) * WIN

        def gather(l, buf, sem):
            pltpu.make_async_copy(
                feat_hbm.at[pl.ds(win_base(l), WIN)], buf, sem).start()

        def wait(l, buf, sem):
            pltpu.make_async_copy(
                feat_hbm.at[pl.ds(win_base(l), WIN)], buf, sem).wait()

        gather(0, fbuf_a, sem_a)
        gather(1, fbuf_b, sem_b)

        pltpu.sync_copy(ids_hbm.at[pl.ds(pb, SLOTS)], idx_all)

        z16 = jnp.zeros((16,), jnp.float32)
        for j in range(NUM_SEGMENTS // 16):
            cnt_buf[pl.ds(j * 16, 16)] = z16

        def zrow(i, carry):
            for j in range(NVEC):
                acc[i, pl.ds(j * 16, 16)] = z16
            return carry

        lax.fori_loop(0, NUM_SEGMENTS, zrow, 0)

        ones16 = jnp.ones((16,), jnp.float32)

        def process(l, buf, sem):
            wait(l, buf, sem)

            @pl.when(l < n_w)
            def _():
                row = shift + l

                def rows(i, carry):
                    ids16 = idx_all[row, pl.ds(i * 16, 16)]
                    plsc.addupdate_scatter(cnt_buf, [ids16], ones16)
                    for lane in range(16):
                        r = i * 16 + lane
                        sid = ids16[lane]
                        for j in range(NVEC):
                            plsc.addupdate(acc.at[sid, pl.ds(j * 16, 16)],
                                           buf[r, pl.ds(j * 16, 16)])
                    return carry

                lax.fori_loop(0, WIN // 16, rows, 0)

        def body(i, carry):
            l0 = 2 * i
            l1 = 2 * i + 1

            process(l0, fbuf_a, sem_a)

            @pl.when(l0 + 2 < SLOTS)
            def _():
                gather(l0 + 2, fbuf_a, sem_a)

            process(l1, fbuf_b, sem_b)

            @pl.when(l1 + 2 < SLOTS)
            def _():
                gather(l1 + 2, fbuf_b, sem_b)

            return carry

        lax.fori_loop(0, SLOTS // 2, body, 0)

        pltpu.sync_copy(acc, out_sum.at[w])
        pltpu.sync_copy(cnt_buf, out_cnt.at[w])

    return seg_sum(feat, ids2d)


def _merge_and_divide(sums, cnts):
    def combine(sum_ref, cnt_ref, out_ref):
        total = jnp.sum(sum_ref[...], axis=0)
        cnt = jnp.sum(cnt_ref[...], axis=0)
        denom = jnp.maximum(cnt, 1.0)[:, None]
        out_ref[...] = total / denom

    return pl.pallas_call(
        combine,
        out_shape=jax.ShapeDtypeStruct((NUM_SEGMENTS, D_FEAT), jnp.float32),
    )(sums, cnts)


@jax.jit
def kernel(feat, segment_ids):
    ids2d = segment_ids.astype(jnp.int32).reshape(NUM_WINDOWS, WIN)
    sums, cnts = _sc_segment_sums(feat, ids2d)
    return _merge_and_divide(sums, cnts)


# speedup vs baseline: 1.8450x; 1.8450x over previous
import functools

import jax
import jax.numpy as jnp
from jax import lax
from jax.experimental import pallas as pl
from jax.experimental.pallas import tpu as pltpu
from jax.experimental.pallas import tpu_sc as plsc

NUM_SEGMENTS = 128
D_FEAT = 128
N_ROWS = 100000

WIN = 80
NC = 2
NS = 16
NW = NC * NS
SLOTS = 20
SC_WINDOWS = NW * SLOTS
SC_ROWS = SC_WINDOWS * WIN

TC_BLOCK = 800
TC_ROWS = N_ROWS - SC_ROWS
TC_STEPS = TC_ROWS // TC_BLOCK
TC_ROW0 = SC_ROWS // TC_BLOCK


def _sc_segment_sums(feat, ids2d):
    mesh = plsc.VectorSubcoreMesh(core_axis_name="c", subcore_axis_name="s")

    @functools.partial(
        pl.kernel,
        out_type=(
            jax.ShapeDtypeStruct((NC, NUM_SEGMENTS, D_FEAT), jnp.float32),
            jax.ShapeDtypeStruct((NW, NUM_SEGMENTS), jnp.float32),
        ),
        mesh=mesh,
        compiler_params=pltpu.CompilerParams(
            use_tc_tiling_on_sc=False, needs_layout_passes=False),
        scratch_types=[
            pltpu.VMEM((WIN, D_FEAT), jnp.float32),
            pltpu.VMEM((WIN, D_FEAT), jnp.float32),
            pltpu.VMEM((SLOTS, WIN), jnp.int32),
            pltpu.VMEM((NUM_SEGMENTS,), jnp.float32),
            pltpu.VMEM_SHARED((NUM_SEGMENTS, D_FEAT), jnp.float32),
            pltpu.SemaphoreType.DMA,
            pltpu.SemaphoreType.DMA,
        ],
    )
    def seg_sum(feat_hbm, ids_hbm, out_sum, out_cnt,
                fbuf_a, fbuf_b, idx_all, cnt_buf, acc_sh, sem_a, sem_b):
        c = lax.axis_index("c")
        s = lax.axis_index("s")
        w = s * NC + c
        b_w = SLOTS * w

        def gather(l, buf, sem):
            pltpu.make_async_copy(
                feat_hbm.at[pl.ds((b_w + l) * WIN, WIN)], buf, sem).start()

        def wait(l, buf, sem):
            pltpu.make_async_copy(
                feat_hbm.at[pl.ds((b_w + l) * WIN, WIN)], buf, sem).wait()

        z16 = jnp.zeros((16,), jnp.float32)
        for j in range(NUM_SEGMENTS // 16):
            cnt_buf[pl.ds(j * 16, 16)] = z16
        rows_per_tile = NUM_SEGMENTS // NS
        for i in range(rows_per_tile):
            for j in range(D_FEAT // 16):
                fbuf_a[i, pl.ds(j * 16, 16)] = z16
        pltpu.sync_copy(fbuf_a.at[pl.ds(0, rows_per_tile)],
                        acc_sh.at[pl.ds(s * rows_per_tile, rows_per_tile)])

        gather(0, fbuf_a, sem_a)
        gather(1, fbuf_b, sem_b)
        pltpu.sync_copy(ids_hbm.at[pl.ds(b_w, SLOTS)], idx_all)
        plsc.subcore_barrier()

        ones16 = jnp.ones((16,), jnp.float32)

        def process(l, buf, sem):
            wait(l, buf, sem)
            row = idx_all.at[l]
            for k in range(WIN // 16):
                ids16 = idx_all[l, pl.ds(k * 16, 16)]
                plsc.addupdate_scatter(cnt_buf, [ids16], ones16)
            pltpu.sync_copy(buf, acc_sh.at[row], add=True)

        def body(i, carry):
            l0 = 2 * i
            l1 = 2 * i + 1

            process(l0, fbuf_a, sem_a)

            @pl.when(l0 + 2 < SLOTS)
            def _():
                gather(l0 + 2, fbuf_a, sem_a)

            process(l1, fbuf_b, sem_b)

            @pl.when(l1 + 2 < SLOTS)
            def _():
                gather(l1 + 2, fbuf_b, sem_b)

            return carry

        lax.fori_loop(0, SLOTS // 2, body, 0)

        pltpu.sync_copy(cnt_buf, out_cnt.at[w])
        plsc.subcore_barrier()

        @pl.when(s == 0)
        def _():
            pltpu.sync_copy(acc_sh, out_sum.at[c])

    return seg_sum(feat, ids2d)


def _tc_segment_sums(feat, ids3d):
    def body(feat_ref, ids_ref, sum_ref, cnt_ref):
        i = pl.program_id(0)
        seg_iota = lax.broadcasted_iota(jnp.int32, (NUM_SEGMENTS, TC_BLOCK), 0)
        onehot = (seg_iota == ids_ref[0]).astype(jnp.float32)
        partial = jnp.dot(onehot, feat_ref[...],
                          preferred_element_type=jnp.float32)
        pcnt = jnp.sum(onehot, axis=1)[None, :]

        @pl.when(i == 0)
        def _():
            sum_ref[...] = partial
            cnt_ref[...] = pcnt

        @pl.when(i > 0)
        def _():
            sum_ref[...] += partial
            cnt_ref[...] += pcnt

    return pl.pallas_call(
        body,
        grid=(TC_STEPS,),
        in_specs=[
            pl.BlockSpec((TC_BLOCK, D_FEAT), lambda i: (TC_ROW0 + i, 0)),
            pl.BlockSpec((1, 1, TC_BLOCK), lambda i: (TC_ROW0 + i, 0, 0)),
        ],
        out_specs=[
            pl.BlockSpec((NUM_SEGMENTS, D_FEAT), lambda i: (0, 0)),
            pl.BlockSpec((1, NUM_SEGMENTS), lambda i: (0, 0)),
        ],
        out_shape=[
            jax.ShapeDtypeStruct((NUM_SEGMENTS, D_FEAT), jnp.float32),
            jax.ShapeDtypeStruct((1, NUM_SEGMENTS), jnp.float32),
        ],
    )(feat, ids3d)


def _merge_and_divide(sc_sums, sc_cnts, tc_sum, tc_cnt):
    def combine(ssum_ref, scnt_ref, tsum_ref, tcnt_ref, out_ref):
        total = ssum_ref[0] + ssum_ref[1] + tsum_ref[...]
        cnt = jnp.sum(scnt_ref[...], axis=0) + tcnt_ref[0]
        denom = jnp.maximum(cnt, 1.0)[:, None]
        out_ref[...] = total / denom

    return pl.pallas_call(
        combine,
        out_shape=jax.ShapeDtypeStruct((NUM_SEGMENTS, D_FEAT), jnp.float32),
    )(sc_sums, sc_cnts, tc_sum, tc_cnt)


@jax.jit
def kernel(feat, segment_ids):
    ids = segment_ids.astype(jnp.int32)
    ids2d = ids.reshape(N_ROWS // WIN, WIN)
    ids3d = ids.reshape(N_ROWS // TC_BLOCK, 1, TC_BLOCK)
    sc_sums, sc_cnts = _sc_segment_sums(feat, ids2d)
    tc_sum, tc_cnt = _tc_segment_sums(feat, ids3d)
    return _merge_and_divide(sc_sums, sc_cnts, tc_sum, tc_cnt)
